# Initial kernel scaffold; baseline (speedup 1.0000x reference)
import jax
import jax.numpy as jnp
from jax.experimental import pallas as pl


def kernel(x, gate_w, e_bias, Wg, Wu, Wd):
    raise NotImplementedError("write your pallas kernel here")


# speedup vs baseline: 1.5403x; 1.5403x over previous
import functools

import jax
import jax.numpy as jnp
from jax import lax
from jax.experimental import pallas as pl
from jax.experimental.pallas import tpu as pltpu

E = 8
TOP_K = 2
D = 1024
FF = 2048
S = 2048
LANES = 128
FBLK = 512
NF = FF // FBLK


def _router_body(x_ref, gw_ref, eb_ref, cf_ref):
    logits = lax.dot_general(x_ref[...], gw_ref[...],
                             (((1,), (1,)), ((), ())),
                             preferred_element_type=jnp.float32)
    scores = jax.nn.sigmoid(logits)
    lane = lax.broadcasted_iota(jnp.int32, (S, LANES), 1)
    valid = lane < E
    neg = jnp.float32(-1e30)
    swb = jnp.where(valid, scores + eb_ref[...], neg)
    m1 = jnp.max(swb, axis=1, keepdims=True)
    i1 = jnp.min(jnp.where(swb == m1, lane, LANES), axis=1, keepdims=True)
    sel1 = lane == i1
    s1 = jnp.sum(jnp.where(sel1, scores, 0.0), axis=1, keepdims=True)
    swb2 = jnp.where(sel1, neg, swb)
    m2 = jnp.max(swb2, axis=1, keepdims=True)
    i2 = jnp.min(jnp.where(swb2 == m2, lane, LANES), axis=1, keepdims=True)
    sel2 = lane == i2
    s2 = jnp.sum(jnp.where(sel2, scores, 0.0), axis=1, keepdims=True)
    denom = s1 + s2 + 1e-20
    cf_ref[...] = (jnp.where(sel1, s1, 0.0) + jnp.where(sel2, s2, 0.0)) / denom


def _ffn_body(cf_ref, x_ref, wg_ref, wu_ref, wd_ref, out_ref, xbf_ref):
    e = pl.program_id(0)
    f = pl.program_id(1)

    @pl.when((e == 0) & (f == 0))
    def _init():
        xbf_ref[...] = x_ref[...].astype(jnp.bfloat16)
        out_ref[...] = jnp.zeros_like(out_ref)

    xbf = xbf_ref[...]
    wg = wg_ref[0].astype(jnp.bfloat16)
    wu = wu_ref[0].astype(jnp.bfloat16)
    g = lax.dot_general(xbf, wg, (((1,), (1,)), ((), ())),
                        preferred_element_type=jnp.float32)
    u = lax.dot_general(xbf, wu, (((1,), (1,)), ((), ())),
                        preferred_element_type=jnp.float32)
    h = (g * jax.nn.sigmoid(g) * u).astype(jnp.bfloat16)
    wd = wd_ref[0].astype(jnp.bfloat16)
    y = lax.dot_general(h, wd, (((1,), (1,)), ((), ())),
                        preferred_element_type=jnp.float32)
    lane = lax.broadcasted_iota(jnp.int32, (S, LANES), 1)
    cfc = jnp.sum(jnp.where(lane == e, cf_ref[...], 0.0), axis=1,
                  keepdims=True)
    out_ref[...] += cfc * y


@jax.jit
def kernel(x, gate_w, e_bias, Wg, Wu, Wd):
    b, s, d = x.shape
    x2 = x.reshape(s, d)
    gwp = jnp.zeros((LANES, D), jnp.float32).at[:E].set(gate_w)
    ebp = jnp.zeros((1, LANES), jnp.float32).at[0, :E].set(e_bias)

    cf = pl.pallas_call(
        _router_body,
        out_shape=jax.ShapeDtypeStruct((S, LANES), jnp.float32),
        in_specs=[
            pl.BlockSpec((S, D), lambda: (0, 0)),
            pl.BlockSpec((LANES, D), lambda: (0, 0)),
            pl.BlockSpec((1, LANES), lambda: (0, 0)),
        ],
        out_specs=pl.BlockSpec((S, LANES), lambda: (0, 0)),
    )(x2, gwp, ebp)

    out = pl.pallas_call(
        _ffn_body,
        grid=(E, NF),
        out_shape=jax.ShapeDtypeStruct((S, D), jnp.float32),
        in_specs=[
            pl.BlockSpec((S, LANES), lambda e, f: (0, 0)),
            pl.BlockSpec((S, D), lambda e, f: (0, 0)),
            pl.BlockSpec((1, FBLK, D), lambda e, f: (e, f, 0)),
            pl.BlockSpec((1, FBLK, D), lambda e, f: (e, f, 0)),
            pl.BlockSpec((1, D, FBLK), lambda e, f: (e, 0, f)),
        ],
        out_specs=pl.BlockSpec((S, D), lambda e, f: (0, 0)),
        scratch_shapes=[pltpu.VMEM((S, D), jnp.bfloat16)],
        compiler_params=pltpu.CompilerParams(
            dimension_semantics=("arbitrary", "arbitrary"),
        ),
    )(cf, x2, Wg, Wu, Wd)

    return out.reshape(b, s, d)
